# baseline (speedup 1.0000x reference)
import functools

import jax
import jax.numpy as jnp
from jax import lax
from jax.experimental import pallas as pl
from jax.experimental.pallas import tpu as pltpu
from jax.experimental.pallas import tpu_sc as plsc

_EMBED_DIM = 128
_D_MODEL = 2 * _EMBED_DIM
_HIDDEN = 2 * _D_MODEL
_STEPS = 4
_SUP_PAD = 8

_NC = 2
_NS = 16
_NW = _NC * _NS


def _sc_gather(table, idx_all):
    n_rows = idx_all.shape[0]
    b_per_w = n_rows // _NW
    mesh = plsc.VectorSubcoreMesh(core_axis_name="c", subcore_axis_name="s")

    @functools.partial(
        pl.kernel,
        mesh=mesh,
        out_type=jax.ShapeDtypeStruct((n_rows, _EMBED_DIM), jnp.float32),
        scratch_types=[
            pltpu.VMEM((b_per_w,), jnp.int32),
            pltpu.VMEM((b_per_w, _EMBED_DIM), jnp.float32),
            pltpu.SemaphoreType.DMA,
        ],
    )
    def gather_kernel(table_hbm, idx_hbm, out_hbm, idx_v, rows_v, sem):
        wid = lax.axis_index("s") * _NC + lax.axis_index("c")
        base = wid * b_per_w
        pltpu.sync_copy(idx_hbm.at[pl.ds(base, b_per_w)], idx_v)
        pltpu.async_copy(table_hbm.at[idx_v], rows_v, sem).wait()
        pltpu.sync_copy(rows_v, out_hbm.at[pl.ds(base, b_per_w)])

    return gather_kernel(table, idx_all)


def _sigmoid(x):
    return 1.0 / (1.0 + jnp.exp(-x))


def _matcher_body(few, qb_ref, sp_ref, p1w_ref, p1b_ref, p2w_ref, p2b_ref,
                  lna_ref, lnb_ref, wcat_ref, whhr_ref, bih_ref, bhh_ref,
                  out_ref):
    f32 = jnp.float32
    bf16 = jnp.bfloat16
    dims = (((1,), (1,)), ((), ()))
    n4h = 4 * _HIDDEN

    s = sp_ref[...]
    h1 = lax.dot_general(s, p1w_ref[...], dims, preferred_element_type=f32)
    h1 = jnp.maximum(h1 + p1b_ref[...], 0.0)
    h2 = lax.dot_general(h1, p2w_ref[...], dims, preferred_element_type=f32)
    z = h2 + p2b_ref[...] + s
    mu = jnp.mean(z, axis=1, keepdims=True)
    zc = z - mu
    var = jnp.sum(zc * zc, axis=1, keepdims=True) / (_D_MODEL - 1)
    sg = zc / (jnp.sqrt(var) + 1e-6) * lna_ref[...] + lnb_ref[...]
    row = lax.broadcasted_iota(jnp.int32, (_SUP_PAD, 1), 0)
    sg = jnp.where(row < few, sg, 0.0)

    s_r = lax.dot_general(sg, whhr_ref[...], dims, preferred_element_type=f32)

    wcat = wcat_ref[...]
    whh_h = wcat[n4h:, :]
    qb = qb_ref[...]
    qb16 = qb.astype(bf16)
    gq = lax.dot_general(qb16, wcat, dims, preferred_element_type=f32)
    g0 = gq[:, :n4h] + bih_ref[...] + bhh_ref[...]
    qh = gq[:, n4h:]
    qs = lax.dot_general(qb, sg, dims, preferred_element_type=f32)

    col = lax.broadcasted_iota(jnp.int32, (1, _SUP_PAD), 1)
    gates = g0
    c = None
    for t in range(_STEPS):
        gi = _sigmoid(gates[:, :_HIDDEN])
        gf = _sigmoid(gates[:, _HIDDEN:2 * _HIDDEN])
        gg = jnp.tanh(gates[:, 2 * _HIDDEN:3 * _HIDDEN])
        go = _sigmoid(gates[:, 3 * _HIDDEN:])
        c = gi * gg if c is None else gf * c + gi * gg
        hc = go * jnp.tanh(c)
        hch = hc[:, :_D_MODEL]
        logits = qs + lax.dot_general(hch, sg, dims, preferred_element_type=f32)
        if t == _STEPS - 1:
            out_ref[...] = logits
        else:
            lm = jnp.where(col < few, logits, -1e30)
            m = jnp.max(lm, axis=1, keepdims=True)
            e = jnp.exp(lm - m)
            attn = e / jnp.sum(e, axis=1, keepdims=True)
            gates = (g0 + qh
                     + lax.dot_general(hch.astype(bf16), whh_h, dims,
                                       preferred_element_type=f32)
                     + jnp.dot(attn, s_r, preferred_element_type=f32))


def _matcher_call(q, s_p, proj1_w, proj1_b, proj2_w, proj2_b, ln_a, ln_b,
                  w_ih, w_hh, b_ih, b_hh, few, blk):
    batch = q.shape[0]
    nb = batch // blk
    wcat = jnp.concatenate([w_ih, w_hh[:, :_D_MODEL]], axis=0).astype(
        jnp.bfloat16)
    whh_r = w_hh[:, _D_MODEL:]
    whole = lambda shape: pl.BlockSpec(shape, lambda i: (0, 0))
    return pl.pallas_call(
        functools.partial(_matcher_body, few),
        grid=(nb,),
        in_specs=[
            pl.BlockSpec((blk, _D_MODEL), lambda i: (i, 0)),
            whole((_SUP_PAD, _D_MODEL)),
            whole(proj1_w.shape),
            whole((1, proj1_b.shape[0])),
            whole(proj2_w.shape),
            whole((1, proj2_b.shape[0])),
            whole((1, ln_a.shape[0])),
            whole((1, ln_b.shape[0])),
            whole(wcat.shape),
            whole(whh_r.shape),
            whole((1, b_ih.shape[0])),
            whole((1, b_hh.shape[0])),
        ],
        out_specs=pl.BlockSpec((blk, _SUP_PAD), lambda i: (i, 0)),
        out_shape=jax.ShapeDtypeStruct((batch, _SUP_PAD), jnp.float32),
        compiler_params=pltpu.CompilerParams(
            dimension_semantics=("arbitrary",)),
    )(q, s_p, proj1_w, proj1_b.reshape(1, -1), proj2_w,
      proj2_b.reshape(1, -1), ln_a.reshape(1, -1), ln_b.reshape(1, -1),
      wcat, whh_r, b_ih.reshape(1, -1), b_hh.reshape(1, -1))


def kernel(query, support, table, proj1_w, proj1_b, proj2_w, proj2_b,
           ln_a, ln_b, w_ih, w_hh, b_ih, b_hh):
    batch = query.shape[0]
    few = support.shape[0]

    qi = query.reshape(-1).astype(jnp.int32)
    si = support.reshape(-1).astype(jnp.int32)
    n_idx = qi.shape[0] + si.shape[0]
    align = 8 * _NW
    n_pad = (-n_idx) % align
    zero_row = table.shape[0] - 1
    idx_all = jnp.concatenate(
        [qi, si, jnp.full((n_pad,), zero_row, jnp.int32)])
    rows = _sc_gather(table, idx_all)

    q = rows[:2 * batch].reshape(batch, _D_MODEL)
    s = rows[2 * batch:2 * batch + 2 * few].reshape(few, _D_MODEL)
    s_p = jnp.pad(s, ((0, _SUP_PAD - few), (0, 0)))

    out_p = _matcher_call(q, s_p, proj1_w, proj1_b, proj2_w, proj2_b,
                          ln_a, ln_b, w_ih, w_hh, b_ih, b_hh, few, blk=512)
    return out_p[:, :few]


# speedup vs baseline: 1.1985x; 1.1985x over previous
import functools

import jax
import jax.numpy as jnp
from jax import lax
from jax.experimental import pallas as pl
from jax.experimental.pallas import tpu as pltpu
from jax.experimental.pallas import tpu_sc as plsc

_EMBED_DIM = 128
_D_MODEL = 2 * _EMBED_DIM
_HIDDEN = 2 * _D_MODEL
_STEPS = 4
_SUP_PAD = 8
_SUP_ROWS = 2 * _SUP_PAD

_NC = 2
_NS = 16
_NW = _NC * _NS


def _sc_gather(table, idx1, idx2, idxs):
    n = idx1.shape[0]
    b_per_w = n // _NS
    mesh = plsc.VectorSubcoreMesh(core_axis_name="c", subcore_axis_name="s")

    @functools.partial(
        pl.kernel,
        mesh=mesh,
        out_type=(
            jax.ShapeDtypeStruct((n, _EMBED_DIM), jnp.float32),
            jax.ShapeDtypeStruct((n, _EMBED_DIM), jnp.float32),
            jax.ShapeDtypeStruct((_SUP_ROWS, _EMBED_DIM), jnp.float32),
        ),
        scratch_types=[
            pltpu.VMEM((b_per_w,), jnp.int32),
            pltpu.VMEM((b_per_w, _EMBED_DIM), jnp.float32),
            pltpu.VMEM((_SUP_ROWS,), jnp.int32),
            pltpu.VMEM((_SUP_ROWS, _EMBED_DIM), jnp.float32),
            pltpu.SemaphoreType.DMA,
        ],
    )
    def gather_kernel(table_hbm, i1_hbm, i2_hbm, is_hbm, o1_hbm, o2_hbm,
                      os_hbm, idx_v, rows_v, idxs_v, rows_s, sem):
        wid = lax.axis_index("s") * _NC + lax.axis_index("c")
        base = (wid % _NS) * b_per_w

        @pl.when(wid < _NS)
        def _():
            pltpu.sync_copy(i1_hbm.at[pl.ds(base, b_per_w)], idx_v)
            pltpu.async_copy(table_hbm.at[idx_v], rows_v, sem).wait()
            pltpu.sync_copy(rows_v, o1_hbm.at[pl.ds(base, b_per_w)])

        @pl.when(wid >= _NS)
        def _():
            pltpu.sync_copy(i2_hbm.at[pl.ds(base, b_per_w)], idx_v)
            pltpu.async_copy(table_hbm.at[idx_v], rows_v, sem).wait()
            pltpu.sync_copy(rows_v, o2_hbm.at[pl.ds(base, b_per_w)])

        @pl.when(wid == _NW - 1)
        def _():
            pltpu.sync_copy(is_hbm, idxs_v)
            pltpu.async_copy(table_hbm.at[idxs_v], rows_s, sem).wait()
            pltpu.sync_copy(rows_s, os_hbm)

    return gather_kernel(table, idx1, idx2, idxs)


def _sigmoid(x):
    return 0.5 + 0.5 * jnp.tanh(0.5 * x)


def _matcher_body(few, qa_ref, qb_ref, sp_ref, p1w_ref, p1b_ref, p2w_ref,
                  p2b_ref, lna_ref, lnb_ref, wcat_ref, whhr_ref, bih_ref,
                  bhh_ref, out_ref):
    f32 = jnp.float32
    dims = (((1,), (1,)), ((), ()))
    n4h = 4 * _HIDDEN

    s = sp_ref[...]
    h1 = lax.dot_general(s, p1w_ref[...], dims, preferred_element_type=f32)
    h1 = jnp.maximum(h1 + p1b_ref[...], 0.0)
    h2 = lax.dot_general(h1, p2w_ref[...], dims, preferred_element_type=f32)
    z = h2 + p2b_ref[...] + s
    mu = jnp.mean(z, axis=1, keepdims=True)
    zc = z - mu
    var = jnp.sum(zc * zc, axis=1, keepdims=True) / (_D_MODEL - 1)
    sg = zc / (jnp.sqrt(var) + 1e-6) * lna_ref[...] + lnb_ref[...]
    row = lax.broadcasted_iota(jnp.int32, (_SUP_PAD, 1), 0)
    sg = jnp.where(row < few, sg, 0.0)

    s_r = lax.dot_general(sg, whhr_ref[...], dims, preferred_element_type=f32)

    wcat = wcat_ref[...]
    whh_h = wcat[n4h:, :]
    qa = qa_ref[...]
    qb = qb_ref[...]
    gq = (lax.dot_general(qa, wcat[:, :_EMBED_DIM], dims,
                          preferred_element_type=f32)
          + lax.dot_general(qb, wcat[:, _EMBED_DIM:], dims,
                            preferred_element_type=f32))
    g0 = gq[:, :n4h] + bih_ref[...] + bhh_ref[...]
    qh = gq[:, n4h:]
    qs = (lax.dot_general(qa, sg[:, :_EMBED_DIM], dims,
                          preferred_element_type=f32)
          + lax.dot_general(qb, sg[:, _EMBED_DIM:], dims,
                            preferred_element_type=f32))

    col = lax.broadcasted_iota(jnp.int32, (1, _SUP_PAD), 1)
    gates = g0
    c = None
    for t in range(_STEPS):
        gi = _sigmoid(gates[:, :_HIDDEN])
        gf = _sigmoid(gates[:, _HIDDEN:2 * _HIDDEN])
        gg = jnp.tanh(gates[:, 2 * _HIDDEN:3 * _HIDDEN])
        go = _sigmoid(gates[:, 3 * _HIDDEN:])
        c = gi * gg if c is None else gf * c + gi * gg
        hc = go * jnp.tanh(c)
        hch = hc[:, :_D_MODEL]
        logits = qs + lax.dot_general(hch, sg, dims, preferred_element_type=f32)
        if t == _STEPS - 1:
            out_ref[...] = logits
        else:
            lm = jnp.where(col < few, logits, -1e30)
            m = jnp.max(lm, axis=1, keepdims=True)
            e = jnp.exp(lm - m)
            attn = e / jnp.sum(e, axis=1, keepdims=True)
            gates = (g0 + qh
                     + lax.dot_general(hch, whh_h, dims,
                                       preferred_element_type=f32)
                     + jnp.dot(attn, s_r, preferred_element_type=f32))


def _matcher_call(q1, q2, s_p, proj1_w, proj1_b, proj2_w, proj2_b, ln_a,
                  ln_b, w_ih, w_hh, b_ih, b_hh, few, blk):
    batch = q1.shape[0]
    nb = batch // blk
    wcat = jnp.concatenate([w_ih, w_hh[:, :_D_MODEL]], axis=0)
    whh_r = w_hh[:, _D_MODEL:]
    whole = lambda shape: pl.BlockSpec(shape, lambda i: (0, 0))
    return pl.pallas_call(
        functools.partial(_matcher_body, few),
        grid=(nb,),
        in_specs=[
            pl.BlockSpec((blk, _EMBED_DIM), lambda i: (i, 0)),
            pl.BlockSpec((blk, _EMBED_DIM), lambda i: (i, 0)),
            whole((_SUP_PAD, _D_MODEL)),
            whole(proj1_w.shape),
            whole((1, proj1_b.shape[0])),
            whole(proj2_w.shape),
            whole((1, proj2_b.shape[0])),
            whole((1, ln_a.shape[0])),
            whole((1, ln_b.shape[0])),
            whole(wcat.shape),
            whole(whh_r.shape),
            whole((1, b_ih.shape[0])),
            whole((1, b_hh.shape[0])),
        ],
        out_specs=pl.BlockSpec((blk, _SUP_PAD), lambda i: (i, 0)),
        out_shape=jax.ShapeDtypeStruct((batch, _SUP_PAD), jnp.float32),
        compiler_params=pltpu.CompilerParams(
            dimension_semantics=("arbitrary",)),
    )(q1, q2, s_p, proj1_w, proj1_b.reshape(1, -1), proj2_w,
      proj2_b.reshape(1, -1), ln_a.reshape(1, -1), ln_b.reshape(1, -1),
      wcat, whh_r, b_ih.reshape(1, -1), b_hh.reshape(1, -1))


def kernel(query, support, table, proj1_w, proj1_b, proj2_w, proj2_b,
           ln_a, ln_b, w_ih, w_hh, b_ih, b_hh):
    batch = query.shape[0]
    few = support.shape[0]
    zero_row = table.shape[0] - 1

    qi = query.astype(jnp.int32)
    idx1 = qi[:, 0]
    idx2 = qi[:, 1]
    idxs = jnp.concatenate(
        [support.reshape(-1).astype(jnp.int32),
         jnp.full((_SUP_ROWS - 2 * few,), zero_row, jnp.int32)])

    q1, q2, srows = _sc_gather(table, idx1, idx2, idxs)
    s_p = srows.reshape(_SUP_PAD, _D_MODEL)

    out_p = _matcher_call(q1, q2, s_p, proj1_w, proj1_b, proj2_w, proj2_b,
                          ln_a, ln_b, w_ih, w_hh, b_ih, b_hh, few, blk=512)
    return out_p[:, :few]


# speedup vs baseline: 1.2694x; 1.0591x over previous
import functools

import jax
import jax.numpy as jnp
from jax import lax
from jax.experimental import pallas as pl
from jax.experimental.pallas import tpu as pltpu
from jax.experimental.pallas import tpu_sc as plsc

_EMBED_DIM = 128
_D_MODEL = 2 * _EMBED_DIM
_HIDDEN = 2 * _D_MODEL
_STEPS = 4
_SUP_PAD = 8
_SUP_ROWS = 2 * _SUP_PAD
_L = 16

_NC = 2
_NS = 16
_NW = _NC * _NS


def _sc_gather(table, idx1, idx2, supidx):
    n = idx1.shape[0]
    b_per_w = n // _NS
    mesh = plsc.VectorSubcoreMesh(core_axis_name="c", subcore_axis_name="s")

    @functools.partial(
        pl.kernel,
        mesh=mesh,
        out_type=(
            jax.ShapeDtypeStruct((n, _EMBED_DIM), jnp.float32),
            jax.ShapeDtypeStruct((n, _EMBED_DIM), jnp.float32),
            jax.ShapeDtypeStruct((_SUP_ROWS, _EMBED_DIM), jnp.float32),
        ),
        scratch_types=[
            pltpu.VMEM((b_per_w,), jnp.int32),
            pltpu.VMEM((b_per_w, _EMBED_DIM), jnp.float32),
            pltpu.VMEM((_SUP_ROWS,), jnp.int32),
            pltpu.VMEM((_SUP_ROWS, _EMBED_DIM), jnp.float32),
            pltpu.SemaphoreType.DMA,
        ],
    )
    def gather_kernel(table_hbm, i1_hbm, i2_hbm, sf_hbm, o1_hbm, o2_hbm,
                      os_hbm, idx_v, rows_v, idxs_v, rows_s, sem):
        wid = lax.axis_index("s") * _NC + lax.axis_index("c")
        half = wid // _NS
        woff = wid % _NS
        base = woff * b_per_w

        @pl.when(half == 0)
        def _():
            pltpu.sync_copy(i1_hbm.at[pl.ds(base, b_per_w)], idx_v)

        @pl.when(half == 1)
        def _():
            pltpu.sync_copy(i2_hbm.at[pl.ds(base, b_per_w)], idx_v)

        pltpu.async_copy(table_hbm.at[idx_v], rows_v, sem).wait()

        @pl.when(half == 0)
        def _():
            pltpu.sync_copy(rows_v, o1_hbm.at[pl.ds(base, b_per_w)])

        @pl.when(half == 1)
        def _():
            pltpu.sync_copy(rows_v, o2_hbm.at[pl.ds(base, b_per_w)])

        @pl.when(wid == _NW - 1)
        def _():
            pltpu.sync_copy(sf_hbm, idxs_v)
            pltpu.async_copy(table_hbm.at[idxs_v], rows_s, sem).wait()
            pltpu.sync_copy(rows_s, os_hbm)

    return gather_kernel(table, idx1, idx2, supidx)


def _sigmoid(x):
    return 0.5 + 0.5 * jnp.tanh(0.5 * x)


def _matcher_body(few, qa_ref, qb_ref, sp_ref, p1w_ref, p1b_ref, p2w_ref,
                  p2b_ref, lna_ref, lnb_ref, wih_ref, whh_ref, bih_ref,
                  bhh_ref, out_ref):
    f32 = jnp.float32
    dims = (((1,), (1,)), ((), ()))
    ed = _EMBED_DIM

    s = sp_ref[...].reshape(_SUP_PAD, _D_MODEL)
    h1 = lax.dot_general(s, p1w_ref[...], dims, preferred_element_type=f32)
    h1 = jnp.maximum(h1 + p1b_ref[...], 0.0)
    h2 = lax.dot_general(h1, p2w_ref[...], dims, preferred_element_type=f32)
    z = h2 + p2b_ref[...] + s
    mu = jnp.mean(z, axis=1, keepdims=True)
    zc = z - mu
    var = jnp.sum(zc * zc, axis=1, keepdims=True) / (_D_MODEL - 1)
    sg = zc / (jnp.sqrt(var) + 1e-6) * lna_ref[...] + lnb_ref[...]
    row = lax.broadcasted_iota(jnp.int32, (_SUP_PAD, 1), 0)
    sg = jnp.where(row < few, sg, 0.0)

    whh = whh_ref[...]
    wih = wih_ref[...]
    whh_h = whh[:, :_D_MODEL]
    s_r = lax.dot_general(sg, whh[:, _D_MODEL:], dims,
                          preferred_element_type=f32)

    qa = qa_ref[...]
    qb = qb_ref[...]
    g0 = (lax.dot_general(qa, wih[:, :ed], dims, preferred_element_type=f32)
          + lax.dot_general(qb, wih[:, ed:], dims, preferred_element_type=f32)
          + bih_ref[...] + bhh_ref[...])
    qh = (lax.dot_general(qa, whh[:, :ed], dims, preferred_element_type=f32)
          + lax.dot_general(qb, whh[:, ed:_D_MODEL], dims,
                            preferred_element_type=f32))
    qs = (lax.dot_general(qa, sg[:, :ed], dims, preferred_element_type=f32)
          + lax.dot_general(qb, sg[:, ed:], dims,
                            preferred_element_type=f32))

    col = lax.broadcasted_iota(jnp.int32, (1, _SUP_PAD), 1)
    gates = g0
    c = None
    for t in range(_STEPS):
        gi = _sigmoid(gates[:, :_HIDDEN])
        gf = _sigmoid(gates[:, _HIDDEN:2 * _HIDDEN])
        gg = jnp.tanh(gates[:, 2 * _HIDDEN:3 * _HIDDEN])
        go = _sigmoid(gates[:, 3 * _HIDDEN:])
        c = gi * gg if c is None else gf * c + gi * gg
        hc = go * jnp.tanh(c)
        hch = hc[:, :_D_MODEL]
        logits = qs + lax.dot_general(hch, sg, dims, preferred_element_type=f32)
        if t == _STEPS - 1:
            out_ref[...] = logits[:, :few]
        else:
            lm = jnp.where(col < few, logits, -1e30)
            m = jnp.max(lm, axis=1, keepdims=True)
            e = jnp.exp(lm - m)
            attn = e / jnp.sum(e, axis=1, keepdims=True)
            gates = (g0 + qh
                     + lax.dot_general(hch, whh_h, dims,
                                       preferred_element_type=f32)
                     + jnp.dot(attn, s_r, preferred_element_type=f32))


def _matcher_call(q1, q2, srows, proj1_w, proj1_b, proj2_w, proj2_b, ln_a,
                  ln_b, w_ih, w_hh, b_ih, b_hh, few, blk):
    batch = q1.shape[0]
    nb = batch // blk
    whole = lambda shape: pl.BlockSpec(shape, lambda i: (0, 0))
    return pl.pallas_call(
        functools.partial(_matcher_body, few),
        grid=(nb,),
        in_specs=[
            pl.BlockSpec((blk, _EMBED_DIM), lambda i: (i, 0)),
            pl.BlockSpec((blk, _EMBED_DIM), lambda i: (i, 0)),
            whole((_SUP_ROWS, _EMBED_DIM)),
            whole(proj1_w.shape),
            whole((1, proj1_b.shape[0])),
            whole(proj2_w.shape),
            whole((1, proj2_b.shape[0])),
            whole((1, ln_a.shape[0])),
            whole((1, ln_b.shape[0])),
            whole(w_ih.shape),
            whole(w_hh.shape),
            whole((1, b_ih.shape[0])),
            whole((1, b_hh.shape[0])),
        ],
        out_specs=pl.BlockSpec((blk, few), lambda i: (i, 0)),
        out_shape=jax.ShapeDtypeStruct((batch, few), jnp.float32),
        compiler_params=pltpu.CompilerParams(
            dimension_semantics=("arbitrary",)),
    )(q1, q2, srows, proj1_w, proj1_b.reshape(1, -1), proj2_w,
      proj2_b.reshape(1, -1), ln_a.reshape(1, -1), ln_b.reshape(1, -1),
      w_ih, w_hh, b_ih.reshape(1, -1), b_hh.reshape(1, -1))


def kernel(query, support, table, proj1_w, proj1_b, proj2_w, proj2_b,
           ln_a, ln_b, w_ih, w_hh, b_ih, b_hh):
    few = support.shape[0]
    zero_row = table.shape[0] - 1
    supidx = jnp.concatenate(
        [support.reshape(-1).astype(jnp.int32),
         jnp.full((_SUP_ROWS - 2 * few,), zero_row, jnp.int32)])
    qi = query.astype(jnp.int32)
    q1, q2, srows = _sc_gather(table, qi[:, 0], qi[:, 1], supidx)
    return _matcher_call(q1, q2, srows, proj1_w, proj1_b, proj2_w, proj2_b,
                         ln_a, ln_b, w_ih, w_hh, b_ih, b_hh, few, blk=512)


# speedup vs baseline: 1.2705x; 1.0009x over previous
import functools

import jax
import jax.numpy as jnp
from jax import lax
from jax.experimental import pallas as pl
from jax.experimental.pallas import tpu as pltpu
from jax.experimental.pallas import tpu_sc as plsc

_EMBED_DIM = 128
_D_MODEL = 2 * _EMBED_DIM
_HIDDEN = 2 * _D_MODEL
_STEPS = 4
_SUP_PAD = 8
_SUP_ROWS = 2 * _SUP_PAD
_NSEL = 4 * _D_MODEL

_NC = 2
_NS = 16
_NW = _NC * _NS


def _sc_gather(table, idx1, idx2, supidx):
    n = idx1.shape[0]
    b_per_w = n // _NS
    mesh = plsc.VectorSubcoreMesh(core_axis_name="c", subcore_axis_name="s")

    @functools.partial(
        pl.kernel,
        mesh=mesh,
        out_type=(
            jax.ShapeDtypeStruct((n, _EMBED_DIM), jnp.float32),
            jax.ShapeDtypeStruct((n, _EMBED_DIM), jnp.float32),
            jax.ShapeDtypeStruct((_SUP_ROWS, _EMBED_DIM), jnp.float32),
        ),
        scratch_types=[
            pltpu.VMEM((b_per_w,), jnp.int32),
            pltpu.VMEM((b_per_w, _EMBED_DIM), jnp.float32),
            pltpu.VMEM((_SUP_ROWS,), jnp.int32),
            pltpu.VMEM((_SUP_ROWS, _EMBED_DIM), jnp.float32),
            pltpu.SemaphoreType.DMA,
        ],
    )
    def gather_kernel(table_hbm, i1_hbm, i2_hbm, sf_hbm, o1_hbm, o2_hbm,
                      os_hbm, idx_v, rows_v, idxs_v, rows_s, sem):
        wid = lax.axis_index("s") * _NC + lax.axis_index("c")
        half = wid // _NS
        woff = wid % _NS
        base = woff * b_per_w

        @pl.when(half == 0)
        def _():
            pltpu.sync_copy(i1_hbm.at[pl.ds(base, b_per_w)], idx_v)

        @pl.when(half == 1)
        def _():
            pltpu.sync_copy(i2_hbm.at[pl.ds(base, b_per_w)], idx_v)

        pltpu.async_copy(table_hbm.at[idx_v], rows_v, sem).wait()

        @pl.when(half == 0)
        def _():
            pltpu.sync_copy(rows_v, o1_hbm.at[pl.ds(base, b_per_w)])

        @pl.when(half == 1)
        def _():
            pltpu.sync_copy(rows_v, o2_hbm.at[pl.ds(base, b_per_w)])

        @pl.when(wid == _NW - 1)
        def _():
            pltpu.sync_copy(sf_hbm, idxs_v)
            pltpu.async_copy(table_hbm.at[idxs_v], rows_s, sem).wait()
            pltpu.sync_copy(rows_s, os_hbm)

    return gather_kernel(table, idx1, idx2, supidx)


def _sigmoid(x):
    return 0.5 + 0.5 * jnp.tanh(0.5 * x)


def _gate_rows(w):
    return jnp.concatenate(
        [w[g * _HIDDEN:g * _HIDDEN + _D_MODEL] for g in range(4)], axis=0)


def _matcher_body(few, qa_ref, qb_ref, sp_ref, p1w_ref, p1b_ref, p2w_ref,
                  p2b_ref, lna_ref, lnb_ref, wihs_ref, whhs_ref, bsel_ref,
                  out_ref):
    f32 = jnp.float32
    dims = (((1,), (1,)), ((), ()))
    ed = _EMBED_DIM
    d = _D_MODEL

    s = sp_ref[...].reshape(_SUP_PAD, _D_MODEL)
    h1 = lax.dot_general(s, p1w_ref[...], dims, preferred_element_type=f32)
    h1 = jnp.maximum(h1 + p1b_ref[...], 0.0)
    h2 = lax.dot_general(h1, p2w_ref[...], dims, preferred_element_type=f32)
    z = h2 + p2b_ref[...] + s
    mu = jnp.mean(z, axis=1, keepdims=True)
    zc = z - mu
    var = jnp.sum(zc * zc, axis=1, keepdims=True) / (_D_MODEL - 1)
    sg = zc / (jnp.sqrt(var) + 1e-6) * lna_ref[...] + lnb_ref[...]
    row = lax.broadcasted_iota(jnp.int32, (_SUP_PAD, 1), 0)
    sg = jnp.where(row < few, sg, 0.0)

    wihs = wihs_ref[...]
    whhs = whhs_ref[...]
    whh_h = whhs[:, :d]
    s_r = lax.dot_general(sg, whhs[:, d:], dims, preferred_element_type=f32)

    qa = qa_ref[...]
    qb = qb_ref[...]
    g0 = (lax.dot_general(qa, wihs[:, :ed], dims, preferred_element_type=f32)
          + lax.dot_general(qb, wihs[:, ed:], dims, preferred_element_type=f32)
          + bsel_ref[...])
    g0qh = (g0
            + lax.dot_general(qa, whh_h[:, :ed], dims,
                              preferred_element_type=f32)
            + lax.dot_general(qb, whh_h[:, ed:], dims,
                              preferred_element_type=f32))
    qs = (lax.dot_general(qa, sg[:, :ed], dims, preferred_element_type=f32)
          + lax.dot_general(qb, sg[:, ed:], dims,
                            preferred_element_type=f32))

    col = lax.broadcasted_iota(jnp.int32, (1, _SUP_PAD), 1)
    gates = g0
    c = None
    for t in range(_STEPS):
        gi = _sigmoid(gates[:, :d])
        gg = jnp.tanh(gates[:, 2 * d:3 * d])
        if c is None:
            c = gi * gg
        else:
            gf = _sigmoid(gates[:, d:2 * d])
            c = gf * c + gi * gg
        go = _sigmoid(gates[:, 3 * d:])
        hch = go * jnp.tanh(c)
        logits = qs + lax.dot_general(hch, sg, dims, preferred_element_type=f32)
        if t == _STEPS - 1:
            out_ref[...] = logits[:, :few]
        else:
            lm = jnp.where(col < few, logits, -1e30)
            m = jnp.max(lm, axis=1, keepdims=True)
            e = jnp.exp(lm - m)
            attn = e / jnp.sum(e, axis=1, keepdims=True)
            gates = (g0qh
                     + lax.dot_general(hch, whh_h, dims,
                                       preferred_element_type=f32)
                     + jnp.dot(attn, s_r, preferred_element_type=f32))


def _matcher_call(q1, q2, srows, proj1_w, proj1_b, proj2_w, proj2_b, ln_a,
                  ln_b, w_ih, w_hh, b_ih, b_hh, few, blk):
    batch = q1.shape[0]
    nb = batch // blk
    wihs = _gate_rows(w_ih)
    whhs = _gate_rows(w_hh)
    bsel = _gate_rows((b_ih + b_hh).reshape(-1, 1)).reshape(1, _NSEL)
    whole = lambda shape: pl.BlockSpec(shape, lambda i: (0, 0))
    return pl.pallas_call(
        functools.partial(_matcher_body, few),
        grid=(nb,),
        in_specs=[
            pl.BlockSpec((blk, _EMBED_DIM), lambda i: (i, 0)),
            pl.BlockSpec((blk, _EMBED_DIM), lambda i: (i, 0)),
            whole((_SUP_ROWS, _EMBED_DIM)),
            whole(proj1_w.shape),
            whole((1, proj1_b.shape[0])),
            whole(proj2_w.shape),
            whole((1, proj2_b.shape[0])),
            whole((1, ln_a.shape[0])),
            whole((1, ln_b.shape[0])),
            whole(wihs.shape),
            whole(whhs.shape),
            whole((1, _NSEL)),
        ],
        out_specs=pl.BlockSpec((blk, few), lambda i: (i, 0)),
        out_shape=jax.ShapeDtypeStruct((batch, few), jnp.float32),
        compiler_params=pltpu.CompilerParams(
            dimension_semantics=("arbitrary",)),
    )(q1, q2, srows, proj1_w, proj1_b.reshape(1, -1), proj2_w,
      proj2_b.reshape(1, -1), ln_a.reshape(1, -1), ln_b.reshape(1, -1),
      wihs, whhs, bsel)


def kernel(query, support, table, proj1_w, proj1_b, proj2_w, proj2_b,
           ln_a, ln_b, w_ih, w_hh, b_ih, b_hh):
    few = support.shape[0]
    zero_row = table.shape[0] - 1
    supidx = jnp.concatenate(
        [support.reshape(-1).astype(jnp.int32),
         jnp.full((_SUP_ROWS - 2 * few,), zero_row, jnp.int32)])
    qi = query.astype(jnp.int32)
    q1, q2, srows = _sc_gather(table, qi[:, 0], qi[:, 1], supidx)
    return _matcher_call(q1, q2, srows, proj1_w, proj1_b, proj2_w, proj2_b,
                         ln_a, ln_b, w_ih, w_hh, b_ih, b_hh, few, blk=512)


# speedup vs baseline: 1.2911x; 1.0163x over previous
import functools

import jax
import jax.numpy as jnp
from jax import lax
from jax.experimental import pallas as pl
from jax.experimental.pallas import tpu as pltpu
from jax.experimental.pallas import tpu_sc as plsc

_EMBED_DIM = 128
_D_MODEL = 2 * _EMBED_DIM
_HIDDEN = 2 * _D_MODEL
_STEPS = 4
_SUP_PAD = 8
_SUP_ROWS = 2 * _SUP_PAD
_NSEL = 4 * _D_MODEL

_NC = 2
_NS = 16
_NW = _NC * _NS


def _sc_gather(table, idx1, idx2, supidx):
    n = idx1.shape[0]
    b_per_w = n // _NS
    mesh = plsc.VectorSubcoreMesh(core_axis_name="c", subcore_axis_name="s")

    @functools.partial(
        pl.kernel,
        mesh=mesh,
        out_type=(
            jax.ShapeDtypeStruct((n, _EMBED_DIM), jnp.float32),
            jax.ShapeDtypeStruct((n, _EMBED_DIM), jnp.float32),
            jax.ShapeDtypeStruct((_SUP_ROWS, _EMBED_DIM), jnp.float32),
        ),
        scratch_types=[
            pltpu.VMEM((b_per_w,), jnp.int32),
            pltpu.VMEM((b_per_w, _EMBED_DIM), jnp.float32),
            pltpu.VMEM((_SUP_ROWS,), jnp.int32),
            pltpu.VMEM((_SUP_ROWS, _EMBED_DIM), jnp.float32),
            pltpu.SemaphoreType.DMA,
        ],
    )
    def gather_kernel(table_hbm, i1_hbm, i2_hbm, sf_hbm, o1_hbm, o2_hbm,
                      os_hbm, idx_v, rows_v, idxs_v, rows_s, sem):
        wid = lax.axis_index("s") * _NC + lax.axis_index("c")
        half = wid // _NS
        woff = wid % _NS
        base = woff * b_per_w

        @pl.when(half == 0)
        def _():
            pltpu.sync_copy(i1_hbm.at[pl.ds(base, b_per_w)], idx_v)

        @pl.when(half == 1)
        def _():
            pltpu.sync_copy(i2_hbm.at[pl.ds(base, b_per_w)], idx_v)

        pltpu.async_copy(table_hbm.at[idx_v], rows_v, sem).wait()

        @pl.when(half == 0)
        def _():
            pltpu.sync_copy(rows_v, o1_hbm.at[pl.ds(base, b_per_w)])

        @pl.when(half == 1)
        def _():
            pltpu.sync_copy(rows_v, o2_hbm.at[pl.ds(base, b_per_w)])

        @pl.when(wid == _NW - 1)
        def _():
            pltpu.sync_copy(sf_hbm, idxs_v)
            pltpu.async_copy(table_hbm.at[idxs_v], rows_s, sem).wait()
            pltpu.sync_copy(rows_s, os_hbm)

    return gather_kernel(table, idx1, idx2, supidx)


def _sigmoid(x):
    return 0.5 + 0.5 * jnp.tanh(0.5 * x)


def _gate_rows(w):
    return jnp.concatenate(
        [w[g * _HIDDEN:g * _HIDDEN + _D_MODEL] for g in range(4)], axis=0)


def _matcher_body(few, qa_ref, qb_ref, sp_ref, p1w_ref, p1b_ref, p2w_ref,
                  p2b_ref, lna_ref, lnb_ref, wihs_ref, whhs_ref, bsel_ref,
                  out_ref):
    f32 = jnp.float32
    dims = (((1,), (1,)), ((), ()))
    ed = _EMBED_DIM
    d = _D_MODEL

    s = sp_ref[...].reshape(_SUP_PAD, _D_MODEL)
    h1 = lax.dot_general(s, p1w_ref[...], dims, preferred_element_type=f32)
    h1 = jnp.maximum(h1 + p1b_ref[...], 0.0)
    h2 = lax.dot_general(h1, p2w_ref[...], dims, preferred_element_type=f32)
    z = h2 + p2b_ref[...] + s
    mu = jnp.mean(z, axis=1, keepdims=True)
    zc = z - mu
    var = jnp.sum(zc * zc, axis=1, keepdims=True) / (_D_MODEL - 1)
    sg = zc / (jnp.sqrt(var) + 1e-6) * lna_ref[...] + lnb_ref[...]
    row = lax.broadcasted_iota(jnp.int32, (_SUP_PAD, 1), 0)
    sg = jnp.where(row < few, sg, 0.0)

    wihs = wihs_ref[...]
    whhs = whhs_ref[...]
    whh_h = whhs[:, :d]
    s_r = lax.dot_general(sg, whhs[:, d:], dims, preferred_element_type=f32)

    qa = qa_ref[...]
    qb = qb_ref[...]
    g0 = (lax.dot_general(qa, wihs[:, :ed], dims, preferred_element_type=f32)
          + lax.dot_general(qb, wihs[:, ed:], dims, preferred_element_type=f32)
          + bsel_ref[...])
    g0qh = (g0
            + lax.dot_general(qa, whh_h[:, :ed], dims,
                              preferred_element_type=f32)
            + lax.dot_general(qb, whh_h[:, ed:], dims,
                              preferred_element_type=f32))
    qs = (lax.dot_general(qa, sg[:, :ed], dims, preferred_element_type=f32)
          + lax.dot_general(qb, sg[:, ed:], dims,
                            preferred_element_type=f32))

    col = lax.broadcasted_iota(jnp.int32, (1, _SUP_PAD), 1)
    gates = g0
    c = None
    for t in range(_STEPS):
        gi = _sigmoid(gates[:, :d])
        gg = jnp.tanh(gates[:, 2 * d:3 * d])
        if c is None:
            c = gi * gg
        else:
            gf = _sigmoid(gates[:, d:2 * d])
            c = gf * c + gi * gg
        go = _sigmoid(gates[:, 3 * d:])
        hch = go * jnp.tanh(c)
        logits = qs + lax.dot_general(hch, sg, dims, preferred_element_type=f32)
        if t == _STEPS - 1:
            out_ref[...] = logits[:, :few]
        else:
            lm = jnp.where(col < few, logits, -1e30)
            m = jnp.max(lm, axis=1, keepdims=True)
            e = jnp.exp(lm - m)
            attn = e / jnp.sum(e, axis=1, keepdims=True)
            gates = (g0qh
                     + lax.dot_general(hch, whh_h, dims,
                                       preferred_element_type=f32)
                     + jnp.dot(attn, s_r, preferred_element_type=f32))


def _matcher_call(q1, q2, srows, proj1_w, proj1_b, proj2_w, proj2_b, ln_a,
                  ln_b, w_ih, w_hh, b_ih, b_hh, few, blk):
    batch = q1.shape[0]
    nb = batch // blk
    wihs = _gate_rows(w_ih)
    whhs = _gate_rows(w_hh)
    bsel = _gate_rows((b_ih + b_hh).reshape(-1, 1)).reshape(1, _NSEL)
    whole = lambda shape: pl.BlockSpec(shape, lambda i: (0, 0))
    return pl.pallas_call(
        functools.partial(_matcher_body, few),
        grid=(nb,),
        in_specs=[
            pl.BlockSpec((blk, _EMBED_DIM), lambda i: (i, 0)),
            pl.BlockSpec((blk, _EMBED_DIM), lambda i: (i, 0)),
            whole((_SUP_ROWS, _EMBED_DIM)),
            whole(proj1_w.shape),
            whole((1, proj1_b.shape[0])),
            whole(proj2_w.shape),
            whole((1, proj2_b.shape[0])),
            whole((1, ln_a.shape[0])),
            whole((1, ln_b.shape[0])),
            whole(wihs.shape),
            whole(whhs.shape),
            whole((1, _NSEL)),
        ],
        out_specs=pl.BlockSpec((blk, few), lambda i: (i, 0)),
        out_shape=jax.ShapeDtypeStruct((batch, few), jnp.float32),
        compiler_params=pltpu.CompilerParams(
            dimension_semantics=("arbitrary",)),
    )(q1, q2, srows, proj1_w, proj1_b.reshape(1, -1), proj2_w,
      proj2_b.reshape(1, -1), ln_a.reshape(1, -1), ln_b.reshape(1, -1),
      wihs, whhs, bsel)


def kernel(query, support, table, proj1_w, proj1_b, proj2_w, proj2_b,
           ln_a, ln_b, w_ih, w_hh, b_ih, b_hh):
    few = support.shape[0]
    zero_row = table.shape[0] - 1
    supidx = jnp.concatenate(
        [support.reshape(-1).astype(jnp.int32),
         jnp.full((_SUP_ROWS - 2 * few,), zero_row, jnp.int32)])
    qi = query.astype(jnp.int32)
    q1, q2, srows = _sc_gather(table, qi[:, 0], qi[:, 1], supidx)
    return _matcher_call(q1, q2, srows, proj1_w, proj1_b, proj2_w, proj2_b,
                         ln_a, ln_b, w_ih, w_hh, b_ih, b_hh, few, blk=1024)


# speedup vs baseline: 1.2915x; 1.0003x over previous
import functools

import jax
import jax.numpy as jnp
from jax import lax
from jax.experimental import pallas as pl
from jax.experimental.pallas import tpu as pltpu
from jax.experimental.pallas import tpu_sc as plsc

_EMBED_DIM = 128
_D_MODEL = 2 * _EMBED_DIM
_HIDDEN = 2 * _D_MODEL
_STEPS = 4
_SUP_PAD = 8
_SUP_ROWS = 2 * _SUP_PAD
_NSEL = 4 * _D_MODEL

_NC = 2
_NS = 16
_NW = _NC * _NS


def _sc_gather(table, idx1, idx2, supidx):
    n = idx1.shape[0]
    b_per_w = n // _NS
    mesh = plsc.VectorSubcoreMesh(core_axis_name="c", subcore_axis_name="s")

    @functools.partial(
        pl.kernel,
        mesh=mesh,
        out_type=(
            jax.ShapeDtypeStruct((n, _EMBED_DIM), jnp.float32),
            jax.ShapeDtypeStruct((n, _EMBED_DIM), jnp.float32),
            jax.ShapeDtypeStruct((_SUP_ROWS, _EMBED_DIM), jnp.float32),
        ),
        scratch_types=[
            pltpu.VMEM((b_per_w,), jnp.int32),
            pltpu.VMEM((b_per_w, _EMBED_DIM), jnp.float32),
            pltpu.VMEM((_SUP_ROWS,), jnp.int32),
            pltpu.VMEM((_SUP_ROWS, _EMBED_DIM), jnp.float32),
            pltpu.SemaphoreType.DMA,
        ],
    )
    def gather_kernel(table_hbm, i1_hbm, i2_hbm, sf_hbm, o1_hbm, o2_hbm,
                      os_hbm, idx_v, rows_v, idxs_v, rows_s, sem):
        wid = lax.axis_index("s") * _NC + lax.axis_index("c")
        half = wid // _NS
        woff = wid % _NS
        base = woff * b_per_w

        @pl.when(half == 0)
        def _():
            pltpu.sync_copy(i1_hbm.at[pl.ds(base, b_per_w)], idx_v)

        @pl.when(half == 1)
        def _():
            pltpu.sync_copy(i2_hbm.at[pl.ds(base, b_per_w)], idx_v)

        pltpu.async_copy(table_hbm.at[idx_v], rows_v, sem).wait()

        @pl.when(half == 0)
        def _():
            pltpu.sync_copy(rows_v, o1_hbm.at[pl.ds(base, b_per_w)])

        @pl.when(half == 1)
        def _():
            pltpu.sync_copy(rows_v, o2_hbm.at[pl.ds(base, b_per_w)])

        @pl.when(wid == _NW - 1)
        def _():
            pltpu.sync_copy(sf_hbm, idxs_v)
            pltpu.async_copy(table_hbm.at[idxs_v], rows_s, sem).wait()
            pltpu.sync_copy(rows_s, os_hbm)

    return gather_kernel(table, idx1, idx2, supidx)


def _sigmoid_pre(x):
    return 0.5 + 0.5 * jnp.tanh(x)


def _gate_rows(w):
    return jnp.concatenate(
        [w[g * _HIDDEN:g * _HIDDEN + _D_MODEL] for g in range(4)], axis=0)


def _matcher_body(few, qa_ref, qb_ref, sp_ref, p1w_ref, p1b_ref, p2w_ref,
                  p2b_ref, lna_ref, lnb_ref, wihs_ref, whhs_ref, bsel_ref,
                  out_ref):
    f32 = jnp.float32
    dims = (((1,), (1,)), ((), ()))
    ed = _EMBED_DIM
    d = _D_MODEL

    s = sp_ref[...].reshape(_SUP_PAD, _D_MODEL)
    h1 = lax.dot_general(s, p1w_ref[...], dims, preferred_element_type=f32)
    h1 = jnp.maximum(h1 + p1b_ref[...], 0.0)
    h2 = lax.dot_general(h1, p2w_ref[...], dims, preferred_element_type=f32)
    z = h2 + p2b_ref[...] + s
    mu = jnp.mean(z, axis=1, keepdims=True)
    zc = z - mu
    var = jnp.sum(zc * zc, axis=1, keepdims=True) / (_D_MODEL - 1)
    sg = zc / (jnp.sqrt(var) + 1e-6) * lna_ref[...] + lnb_ref[...]
    row = lax.broadcasted_iota(jnp.int32, (_SUP_PAD, 1), 0)
    sg = jnp.where(row < few, sg, 0.0)

    wihs = wihs_ref[...]
    whhs = whhs_ref[...]
    whh_h = whhs[:, :d]
    s_r = lax.dot_general(sg, whhs[:, d:], dims, preferred_element_type=f32)

    qa = qa_ref[...]
    qb = qb_ref[...]
    g0 = (lax.dot_general(qa, wihs[:, :ed], dims, preferred_element_type=f32)
          + lax.dot_general(qb, wihs[:, ed:], dims, preferred_element_type=f32)
          + bsel_ref[...])
    g0qh = (g0
            + lax.dot_general(qa, whh_h[:, :ed], dims,
                              preferred_element_type=f32)
            + lax.dot_general(qb, whh_h[:, ed:], dims,
                              preferred_element_type=f32))
    qs = (lax.dot_general(qa, sg[:, :ed], dims, preferred_element_type=f32)
          + lax.dot_general(qb, sg[:, ed:], dims,
                            preferred_element_type=f32))

    col = lax.broadcasted_iota(jnp.int32, (1, _SUP_PAD), 1)
    gates = g0
    c = None
    for t in range(_STEPS):
        gi = _sigmoid_pre(gates[:, :d])
        gg = jnp.tanh(gates[:, 2 * d:3 * d])
        if c is None:
            c = gi * gg
        else:
            gf = _sigmoid_pre(gates[:, d:2 * d])
            c = gf * c + gi * gg
        go = _sigmoid_pre(gates[:, 3 * d:])
        hch = go * jnp.tanh(c)
        logits = qs + lax.dot_general(hch, sg, dims, preferred_element_type=f32)
        if t == _STEPS - 1:
            out_ref[...] = logits[:, :few]
        else:
            lm = jnp.where(col < few, logits, -1e30)
            m = jnp.max(lm, axis=1, keepdims=True)
            e = jnp.exp(lm - m)
            attn = e / jnp.sum(e, axis=1, keepdims=True)
            gates = (g0qh
                     + lax.dot_general(hch, whh_h, dims,
                                       preferred_element_type=f32)
                     + jnp.dot(attn, s_r, preferred_element_type=f32))


def _matcher_call(q1, q2, srows, proj1_w, proj1_b, proj2_w, proj2_b, ln_a,
                  ln_b, w_ih, w_hh, b_ih, b_hh, few, blk):
    batch = q1.shape[0]
    nb = batch // blk
    gscale = jnp.concatenate(
        [jnp.full((_D_MODEL, 1), 0.5 if g != 2 else 1.0, jnp.float32)
         for g in range(4)], axis=0)
    wihs = _gate_rows(w_ih) * gscale
    whhs = _gate_rows(w_hh) * gscale
    bsel = (_gate_rows((b_ih + b_hh).reshape(-1, 1))
            * gscale).reshape(1, _NSEL)
    whole = lambda shape: pl.BlockSpec(shape, lambda i: (0, 0))
    return pl.pallas_call(
        functools.partial(_matcher_body, few),
        grid=(nb,),
        in_specs=[
            pl.BlockSpec((blk, _EMBED_DIM), lambda i: (i, 0)),
            pl.BlockSpec((blk, _EMBED_DIM), lambda i: (i, 0)),
            whole((_SUP_ROWS, _EMBED_DIM)),
            whole(proj1_w.shape),
            whole((1, proj1_b.shape[0])),
            whole(proj2_w.shape),
            whole((1, proj2_b.shape[0])),
            whole((1, ln_a.shape[0])),
            whole((1, ln_b.shape[0])),
            whole(wihs.shape),
            whole(whhs.shape),
            whole((1, _NSEL)),
        ],
        out_specs=pl.BlockSpec((blk, few), lambda i: (i, 0)),
        out_shape=jax.ShapeDtypeStruct((batch, few), jnp.float32),
        compiler_params=pltpu.CompilerParams(
            dimension_semantics=("arbitrary",)),
    )(q1, q2, srows, proj1_w, proj1_b.reshape(1, -1), proj2_w,
      proj2_b.reshape(1, -1), ln_a.reshape(1, -1), ln_b.reshape(1, -1),
      wihs, whhs, bsel)


def kernel(query, support, table, proj1_w, proj1_b, proj2_w, proj2_b,
           ln_a, ln_b, w_ih, w_hh, b_ih, b_hh):
    few = support.shape[0]
    zero_row = table.shape[0] - 1
    supidx = jnp.concatenate(
        [support.reshape(-1).astype(jnp.int32),
         jnp.full((_SUP_ROWS - 2 * few,), zero_row, jnp.int32)])
    qi = query.astype(jnp.int32)
    q1, q2, srows = _sc_gather(table, qi[:, 0], qi[:, 1], supidx)
    return _matcher_call(q1, q2, srows, proj1_w, proj1_b, proj2_w, proj2_b,
                         ln_a, ln_b, w_ih, w_hh, b_ih, b_hh, few, blk=1024)
